# baseline (speedup 1.0000x reference)
import functools

import jax
import jax.numpy as jnp
import numpy as np
from jax import lax
from jax.experimental import pallas as pl
from jax.experimental.pallas import tpu as pltpu
from jax.experimental.pallas import tpu_sc as plsc


@functools.lru_cache(maxsize=None)
def _slow_gather_kernel(C, T, H, W, n_slow, hch, nw):
    mesh = plsc.VectorSubcoreMesh(core_axis_name="c", subcore_axis_name="s")
    units = C * n_slow * hch
    per_w = units // nw
    hblk = H // hch
    nbuf = 2

    @functools.partial(
        pl.kernel,
        out_type=jax.ShapeDtypeStruct((C, n_slow, H, W), jnp.float32),
        mesh=mesh,
        scratch_types=[
            pltpu.VMEM((nbuf, hblk, W), jnp.float32),
            pltpu.SemaphoreType.DMA,
            pltpu.SemaphoreType.DMA,
        ],
        cost_estimate=pl.CostEstimate(
            flops=0, transcendentals=0,
            bytes_accessed=2 * C * n_slow * H * W * 4),
    )
    def k(in_hbm, out_hbm, buf, sem_r, sem_w):
        wid = lax.axis_index("s") * 2 + lax.axis_index("c")

        def unit(i):
            j = wid * per_w + i
            part = lax.rem(j, hch)
            r = lax.div(j, hch)
            c = lax.div(r, n_slow)
            t = lax.rem(r, n_slow)
            src_t = lax.div(t * (T - 1), n_slow - 1)
            h0 = part * hblk
            return c, src_t, t, h0

        us = [unit(i) for i in range(per_w)]
        rd = [pltpu.async_copy(
                  in_hbm.at[us[i][0], us[i][1], pl.ds(us[i][3], hblk)],
                  buf.at[i], sem_r)
              for i in range(min(nbuf, per_w))]
        wr = [None] * per_w
        for i in range(per_w):
            c, src_t, t, h0 = us[i]
            rd[i].wait()
            wr[i] = pltpu.async_copy(
                buf.at[i % nbuf], out_hbm.at[c, t, pl.ds(h0, hblk)], sem_w)
            if i + nbuf < per_w:
                wr[i].wait()
                cn, srcn, tn, h0n = us[i + nbuf]
                rd.append(pltpu.async_copy(
                    in_hbm.at[cn, srcn, pl.ds(h0n, hblk)],
                    buf.at[i % nbuf], sem_r))
        for i in range(max(0, per_w - nbuf), per_w):
            wr[i].wait()

    return k


def kernel(frames):
    C, T, H, W = frames.shape
    alpha = 4
    n_slow = T // alpha
    idx_ref = np.linspace(0, T - 1, n_slow).astype(np.int64)
    idx_arith = (np.arange(n_slow) * (T - 1)) // (n_slow - 1)
    assert (idx_ref == idx_arith).all()

    info = plsc.get_sparse_core_info()
    nw = info.num_cores * info.num_subcores

    hch = 1
    while ((C * n_slow * hch) % nw != 0 or H % hch != 0
           or 2 * (H // hch) * W * 4 > 524284):
        hch *= 2

    k = _slow_gather_kernel(C, T, H, W, n_slow, hch, nw)
    slow = k(frames)
    fast = _fast_copy(frames)
    return (slow, fast)


@functools.lru_cache(maxsize=None)
def _fast_copy_kernel(C, T, H, W, tblk):
    def body(in_ref, out_ref, sem):
        cps = [pltpu.make_async_copy(
                   in_ref.at[c, pl.ds(t * tblk, tblk)],
                   out_ref.at[c, pl.ds(t * tblk, tblk)], sem)
               for c in range(C) for t in range(T // tblk)]
        for cp in cps:
            cp.start()
        for cp in cps:
            cp.wait()

    return pl.pallas_call(
        body,
        in_specs=[pl.BlockSpec(memory_space=pl.ANY)],
        out_specs=pl.BlockSpec(memory_space=pl.ANY),
        out_shape=jax.ShapeDtypeStruct((C, T, H, W), jnp.float32),
        scratch_shapes=[pltpu.SemaphoreType.DMA],
    )


def _fast_copy(frames):
    C, T, H, W = frames.shape
    return _fast_copy_kernel(C, T, H, W, 16)(frames)


# speedup vs baseline: 31.3590x; 31.3590x over previous
import functools

import jax
import jax.numpy as jnp
import numpy as np
from jax import lax
from jax.experimental import pallas as pl
from jax.experimental.pallas import tpu as pltpu
from jax.experimental.pallas import tpu_sc as plsc


@functools.lru_cache(maxsize=None)
def _slow_gather_kernel(C, T, H, W, n_slow, hch, nw):
    mesh = plsc.VectorSubcoreMesh(core_axis_name="c", subcore_axis_name="s")
    units = C * n_slow * hch
    per_w = units // nw
    hblk = H // hch
    nbuf = 2

    @functools.partial(
        pl.kernel,
        out_type=jax.ShapeDtypeStruct((C, n_slow, H, W), jnp.float32),
        mesh=mesh,
        scratch_types=[
            pltpu.VMEM((nbuf, hblk, W), jnp.float32),
            pltpu.SemaphoreType.DMA,
            pltpu.SemaphoreType.DMA,
        ],
        cost_estimate=pl.CostEstimate(
            flops=0, transcendentals=0,
            bytes_accessed=2 * C * n_slow * H * W * 4),
    )
    def k(in_hbm, out_hbm, buf, sem_r, sem_w):
        wid = lax.axis_index("s") * 2 + lax.axis_index("c")

        def unit(i):
            j = wid * per_w + i
            part = lax.rem(j, hch)
            r = lax.div(j, hch)
            c = lax.div(r, n_slow)
            t = lax.rem(r, n_slow)
            src_t = lax.div(t * (T - 1), n_slow - 1)
            h0 = part * hblk
            return c, src_t, t, h0

        us = [unit(i) for i in range(per_w)]
        rd = [pltpu.async_copy(
                  in_hbm.at[us[i][0], us[i][1], pl.ds(us[i][3], hblk)],
                  buf.at[i], sem_r)
              for i in range(min(nbuf, per_w))]
        wr = [None] * per_w
        for i in range(per_w):
            c, src_t, t, h0 = us[i]
            rd[i].wait()
            wr[i] = pltpu.async_copy(
                buf.at[i % nbuf], out_hbm.at[c, t, pl.ds(h0, hblk)], sem_w)
            if i + nbuf < per_w:
                wr[i].wait()
                cn, srcn, tn, h0n = us[i + nbuf]
                rd.append(pltpu.async_copy(
                    in_hbm.at[cn, srcn, pl.ds(h0n, hblk)],
                    buf.at[i % nbuf], sem_r))
        for i in range(max(0, per_w - nbuf), per_w):
            wr[i].wait()

    return k


def kernel(frames):
    C, T, H, W = frames.shape
    alpha = 4
    n_slow = T // alpha
    idx_ref = np.linspace(0, T - 1, n_slow).astype(np.int64)
    idx_arith = (np.arange(n_slow) * (T - 1)) // (n_slow - 1)
    assert (idx_ref == idx_arith).all()

    info = plsc.get_sparse_core_info()
    nw = info.num_cores * info.num_subcores

    hch = 1
    while ((C * n_slow * hch) % nw != 0 or H % hch != 0
           or 2 * (H // hch) * W * 4 > 524284):
        hch *= 2

    k = _slow_gather_kernel(C, T, H, W, n_slow, hch, nw)
    slow = k(frames)
    fast = _fast_copy(frames)
    return (slow, fast)


def _fast_copy(frames):
    return lax.max(frames, jnp.float32(-jnp.inf))


# speedup vs baseline: 31.8913x; 1.0170x over previous
import functools

import jax
import jax.numpy as jnp
import numpy as np
from jax import lax
from jax.experimental import pallas as pl
from jax.experimental.pallas import tpu as pltpu
from jax.experimental.pallas import tpu_sc as plsc


@functools.lru_cache(maxsize=None)
def _slow_gather_kernel(C, T, H, W, n_slow, hch, nw):
    mesh = plsc.VectorSubcoreMesh(core_axis_name="c", subcore_axis_name="s")
    units = C * n_slow * hch
    per_w = units // nw
    hblk = H // hch
    nbuf = 2

    @functools.partial(
        pl.kernel,
        out_type=jax.ShapeDtypeStruct((C, n_slow, H, W), jnp.float32),
        mesh=mesh,
        scratch_types=[
            pltpu.VMEM((nbuf, hblk, W), jnp.float32),
            pltpu.SemaphoreType.DMA,
            pltpu.SemaphoreType.DMA,
        ],
        cost_estimate=pl.CostEstimate(
            flops=0, transcendentals=0,
            bytes_accessed=2 * C * n_slow * H * W * 4),
    )
    def k(in_hbm, out_hbm, buf, sem_r, sem_w):
        wid = lax.axis_index("s") * 2 + lax.axis_index("c")

        def unit(i):
            j = wid * per_w + i
            part = lax.rem(j, hch)
            r = lax.div(j, hch)
            c = lax.div(r, n_slow)
            t = lax.rem(r, n_slow)
            src_t = lax.div(t * (T - 1), n_slow - 1)
            h0 = part * hblk
            return c, src_t, t, h0

        us = [unit(i) for i in range(per_w)]
        rd = [pltpu.async_copy(
                  in_hbm.at[us[i][0], us[i][1], pl.ds(us[i][3], hblk)],
                  buf.at[i], sem_r)
              for i in range(min(nbuf, per_w))]
        wr = [None] * per_w
        for i in range(per_w):
            c, src_t, t, h0 = us[i]
            rd[i].wait()
            wr[i] = pltpu.async_copy(
                buf.at[i % nbuf], out_hbm.at[c, t, pl.ds(h0, hblk)], sem_w)
            if i + nbuf < per_w:
                wr[i].wait()
                cn, srcn, tn, h0n = us[i + nbuf]
                rd.append(pltpu.async_copy(
                    in_hbm.at[cn, srcn, pl.ds(h0n, hblk)],
                    buf.at[i % nbuf], sem_r))
        for i in range(max(0, per_w - nbuf), per_w):
            wr[i].wait()

    return k


def kernel(frames):
    C, T, H, W = frames.shape
    alpha = 4
    n_slow = T // alpha
    idx_ref = np.linspace(0, T - 1, n_slow).astype(np.int64)
    idx_arith = (np.arange(n_slow) * (T - 1)) // (n_slow - 1)
    assert (idx_ref == idx_arith).all()

    info = plsc.get_sparse_core_info()
    nw = info.num_cores * info.num_subcores

    hch = 1
    while ((C * n_slow * hch) % nw != 0 or H % hch != 0
           or 2 * (H // hch) * W * 4 > 524284):
        hch *= 2

    k = _slow_gather_kernel(C, T, H, W, n_slow, hch, nw)
    slow = k(frames)
    fast = _fast_copy(frames)
    return (slow, fast)


@functools.lru_cache(maxsize=None)
def _fast_copy_kernel(C, T, H, W, nbuf, grp):
    N = C * T

    def body(in_ref, out_ref, buf, sem_in, sem_out):
        def cp_in(u):
            return pltpu.make_async_copy(
                in_ref.at[u // T, u % T], buf.at[u % nbuf], sem_in.at[u % nbuf])

        def cp_out(u):
            return pltpu.make_async_copy(
                buf.at[u % nbuf], out_ref.at[u // T, u % T], sem_out.at[u % nbuf])

        ngrp = N // grp
        for j in range(grp):
            cp_in(j).start()
        for k in range(ngrp):
            if k + 1 < ngrp:
                for j in range(grp):
                    cp_in((k + 1) * grp + j).start()
            for j in range(grp):
                u = k * grp + j
                cp_in(u).wait()
                cp_out(u).start()
            for j in range(grp):
                cp_out(k * grp + j).wait()

    return pl.pallas_call(
        body,
        in_specs=[pl.BlockSpec(memory_space=pl.ANY)],
        out_specs=pl.BlockSpec(memory_space=pl.ANY),
        out_shape=jax.ShapeDtypeStruct((C, T, H, W), jnp.float32),
        scratch_shapes=[
            pltpu.VMEM((nbuf, H, W), jnp.float32),
            pltpu.SemaphoreType.DMA((nbuf,)),
            pltpu.SemaphoreType.DMA((nbuf,)),
        ],
    )


def _fast_copy(frames):
    C, T, H, W = frames.shape
    return _fast_copy_kernel(C, T, H, W, 16, 8)(frames)


# speedup vs baseline: 32.3518x; 1.0144x over previous
import functools

import jax
import jax.numpy as jnp
import numpy as np
from jax import lax
from jax.experimental import pallas as pl
from jax.experimental.pallas import tpu as pltpu
from jax.experimental.pallas import tpu_sc as plsc


@functools.lru_cache(maxsize=None)
def _slow_gather_kernel(C, T, H, W, n_slow, hch, nw):
    mesh = plsc.VectorSubcoreMesh(core_axis_name="c", subcore_axis_name="s")
    units = C * n_slow * hch
    per_w = units // nw
    hblk = H // hch
    nbuf = 2

    @functools.partial(
        pl.kernel,
        out_type=jax.ShapeDtypeStruct((C, n_slow, H, W), jnp.float32),
        mesh=mesh,
        scratch_types=[
            pltpu.VMEM((nbuf, hblk, W), jnp.float32),
            pltpu.SemaphoreType.DMA,
            pltpu.SemaphoreType.DMA,
        ],
        cost_estimate=pl.CostEstimate(
            flops=0, transcendentals=0,
            bytes_accessed=2 * C * n_slow * H * W * 4),
    )
    def k(in_hbm, out_hbm, buf, sem_r, sem_w):
        wid = lax.axis_index("s") * 2 + lax.axis_index("c")

        def unit(i):
            j = wid * per_w + i
            part = lax.rem(j, hch)
            r = lax.div(j, hch)
            c = lax.div(r, n_slow)
            t = lax.rem(r, n_slow)
            src_t = lax.div(t * (T - 1), n_slow - 1)
            h0 = part * hblk
            return c, src_t, t, h0

        us = [unit(i) for i in range(per_w)]
        rd = [pltpu.async_copy(
                  in_hbm.at[us[i][0], us[i][1], pl.ds(us[i][3], hblk)],
                  buf.at[i], sem_r)
              for i in range(min(nbuf, per_w))]
        wr = [None] * per_w
        for i in range(per_w):
            c, src_t, t, h0 = us[i]
            rd[i].wait()
            wr[i] = pltpu.async_copy(
                buf.at[i % nbuf], out_hbm.at[c, t, pl.ds(h0, hblk)], sem_w)
            if i + nbuf < per_w:
                wr[i].wait()
                cn, srcn, tn, h0n = us[i + nbuf]
                rd.append(pltpu.async_copy(
                    in_hbm.at[cn, srcn, pl.ds(h0n, hblk)],
                    buf.at[i % nbuf], sem_r))
        for i in range(max(0, per_w - nbuf), per_w):
            wr[i].wait()

    return k


def kernel(frames):
    C, T, H, W = frames.shape
    alpha = 4
    n_slow = T // alpha
    idx_ref = np.linspace(0, T - 1, n_slow).astype(np.int64)
    idx_arith = (np.arange(n_slow) * (T - 1)) // (n_slow - 1)
    assert (idx_ref == idx_arith).all()

    info = plsc.get_sparse_core_info()
    nw = info.num_cores * info.num_subcores

    hch = 1
    while ((C * n_slow * hch) % nw != 0 or H % hch != 0
           or 2 * (H // hch) * W * 4 > 524284):
        hch *= 2

    k = _slow_gather_kernel(C, T, H, W, n_slow, hch, nw)
    slow = k(frames)
    fast = _fast_copy(frames)
    return (slow, fast)


@functools.lru_cache(maxsize=None)
def _fast_copy_kernel(C, T, H, W, nbuf, grp):
    N = C * T

    def body(in_ref, out_ref, buf, sem_in, sem_out):
        def cp_in(u):
            return pltpu.make_async_copy(
                in_ref.at[u // T, u % T], buf.at[u % nbuf], sem_in.at[u % nbuf])

        def cp_out(u):
            return pltpu.make_async_copy(
                buf.at[u % nbuf], out_ref.at[u // T, u % T], sem_out.at[u % nbuf])

        ngrp = N // grp
        for j in range(grp):
            cp_in(j).start()
        for k in range(ngrp):
            if k + 1 < ngrp:
                for j in range(grp):
                    cp_in((k + 1) * grp + j).start()
            for j in range(grp):
                u = k * grp + j
                cp_in(u).wait()
                cp_out(u).start()
            for j in range(grp):
                cp_out(k * grp + j).wait()

    return pl.pallas_call(
        body,
        in_specs=[pl.BlockSpec(memory_space=pl.ANY)],
        out_specs=pl.BlockSpec(memory_space=pl.ANY),
        out_shape=jax.ShapeDtypeStruct((C, T, H, W), jnp.float32),
        scratch_shapes=[
            pltpu.VMEM((nbuf, H, W), jnp.float32),
            pltpu.SemaphoreType.DMA((nbuf,)),
            pltpu.SemaphoreType.DMA((nbuf,)),
        ],
    )


def _fast_copy(frames):
    C, T, H, W = frames.shape
    return _fast_copy_kernel(C, T, H, W, 32, 16)(frames)


# speedup vs baseline: 32.9551x; 1.0186x over previous
import functools

import jax
import jax.numpy as jnp
import numpy as np
from jax import lax
from jax.experimental import pallas as pl
from jax.experimental.pallas import tpu as pltpu
from jax.experimental.pallas import tpu_sc as plsc


@functools.lru_cache(maxsize=None)
def _slow_gather_kernel(C, T, H, W, n_slow, hch, nw):
    mesh = plsc.VectorSubcoreMesh(core_axis_name="c", subcore_axis_name="s")
    units = C * n_slow * hch
    per_w = units // nw
    hblk = H // hch
    nbuf = 2

    @functools.partial(
        pl.kernel,
        out_type=jax.ShapeDtypeStruct((C, n_slow, H, W), jnp.float32),
        mesh=mesh,
        scratch_types=[
            pltpu.VMEM((nbuf, hblk, W), jnp.float32),
            pltpu.SemaphoreType.DMA,
            pltpu.SemaphoreType.DMA,
        ],
        cost_estimate=pl.CostEstimate(
            flops=0, transcendentals=0,
            bytes_accessed=2 * C * n_slow * H * W * 4),
    )
    def k(in_hbm, out_hbm, buf, sem_r, sem_w):
        wid = lax.axis_index("s") * 2 + lax.axis_index("c")

        def unit(i):
            j = wid * per_w + i
            part = lax.rem(j, hch)
            r = lax.div(j, hch)
            c = lax.div(r, n_slow)
            t = lax.rem(r, n_slow)
            src_t = lax.div(t * (T - 1), n_slow - 1)
            h0 = part * hblk
            return c, src_t, t, h0

        us = [unit(i) for i in range(per_w)]
        rd = [pltpu.async_copy(
                  in_hbm.at[us[i][0], us[i][1], pl.ds(us[i][3], hblk)],
                  buf.at[i], sem_r)
              for i in range(min(nbuf, per_w))]
        wr = [None] * per_w
        for i in range(per_w):
            c, src_t, t, h0 = us[i]
            rd[i].wait()
            wr[i] = pltpu.async_copy(
                buf.at[i % nbuf], out_hbm.at[c, t, pl.ds(h0, hblk)], sem_w)
            if i + nbuf < per_w:
                wr[i].wait()
                cn, srcn, tn, h0n = us[i + nbuf]
                rd.append(pltpu.async_copy(
                    in_hbm.at[cn, srcn, pl.ds(h0n, hblk)],
                    buf.at[i % nbuf], sem_r))
        for i in range(max(0, per_w - nbuf), per_w):
            wr[i].wait()

    return k


def kernel(frames):
    C, T, H, W = frames.shape
    alpha = 4
    n_slow = T // alpha
    idx_ref = np.linspace(0, T - 1, n_slow).astype(np.int64)
    idx_arith = (np.arange(n_slow) * (T - 1)) // (n_slow - 1)
    assert (idx_ref == idx_arith).all()

    info = plsc.get_sparse_core_info()
    nw = info.num_cores * info.num_subcores

    hch = 1
    while ((C * n_slow * hch) % nw != 0 or H % hch != 0
           or 2 * (H // hch) * W * 4 > 524284):
        hch *= 2

    k = _slow_gather_kernel(C, T, H, W, n_slow, hch, nw)
    slow = k(frames)
    fast = _fast_copy(frames)
    return (slow, fast)


@functools.lru_cache(maxsize=None)
def _fast_copy_kernel(C, T, H, W, nbuf, grp, tsub):
    N = C * (T // tsub)
    TS = T // tsub

    def body(in_ref, out_ref, buf, sem_in, sem_out):
        def cp_in(u):
            return pltpu.make_async_copy(
                in_ref.at[u // TS, pl.ds((u % TS) * tsub, tsub)],
                buf.at[u % nbuf], sem_in.at[u % nbuf])

        def cp_out(u):
            return pltpu.make_async_copy(
                buf.at[u % nbuf],
                out_ref.at[u // TS, pl.ds((u % TS) * tsub, tsub)],
                sem_out.at[u % nbuf])

        ngrp = N // grp
        for j in range(grp):
            cp_in(j).start()
        for k in range(ngrp):
            if k + 1 < ngrp:
                for j in range(grp):
                    cp_in((k + 1) * grp + j).start()
            for j in range(grp):
                u = k * grp + j
                cp_in(u).wait()
                cp_out(u).start()
            for j in range(grp):
                cp_out(k * grp + j).wait()

    return pl.pallas_call(
        body,
        in_specs=[pl.BlockSpec(memory_space=pl.ANY)],
        out_specs=pl.BlockSpec(memory_space=pl.ANY),
        out_shape=jax.ShapeDtypeStruct((C, T, H, W), jnp.float32),
        scratch_shapes=[
            pltpu.VMEM((nbuf, tsub, H, W), jnp.float32),
            pltpu.SemaphoreType.DMA((nbuf,)),
            pltpu.SemaphoreType.DMA((nbuf,)),
        ],
    )


def _fast_copy(frames):
    C, T, H, W = frames.shape
    return _fast_copy_kernel(C, T, H, W, 8, 4, 8)(frames)
